# Initial kernel scaffold; baseline (speedup 1.0000x reference)
import jax
import jax.numpy as jnp
from jax.experimental import pallas as pl


def kernel(text, emb_table, W, b):
    raise NotImplementedError("write your pallas kernel here")


# speedup vs baseline: 30.5817x; 30.5817x over previous
import functools

import jax
import jax.numpy as jnp
from jax import lax
from jax.experimental import pallas as pl
from jax.experimental.pallas import tpu as pltpu
from jax.experimental.pallas import tpu_sc as plsc

VOCAB = 100000
EMBED = 128
BATCH = 4096
SEQ = 200

TC_BLOCK = 2048
N_BLOCKS = -(-VOCAB // TC_BLOCK)
SCORES_PAD = N_BLOCKS * TC_BLOCK


def _scores_body(wt_ref, b_ref, emb_ref, out_ref):
    s = lax.dot_general(
        wt_ref[...], emb_ref[...],
        dimension_numbers=(((1,), (1,)), ((), ())),
        preferred_element_type=jnp.float32,
    )
    out_ref[...] = (s + b_ref[0, 0]) * (1.0 / SEQ)


def _compute_scores(emb_table, wt, b2):
    return pl.pallas_call(
        _scores_body,
        grid=(N_BLOCKS,),
        in_specs=[
            pl.BlockSpec((1, EMBED), lambda i: (0, 0)),
            pl.BlockSpec((1, 1), lambda i: (0, 0)),
            pl.BlockSpec((TC_BLOCK, EMBED), lambda i: (i, 0)),
        ],
        out_specs=pl.BlockSpec((1, TC_BLOCK), lambda i: (0, i)),
        out_shape=jax.ShapeDtypeStruct((1, SCORES_PAD), jnp.float32),
    )(wt, b2, emb_table)


NUM_WORKERS = 32
ROWS_PER = BATCH // NUM_WORKERS
IDX_PER = ROWS_PER * SEQ
LANES = 16
GROUPS = ROWS_PER // LANES

_mesh = plsc.VectorSubcoreMesh(core_axis_name="c", subcore_axis_name="s")


@functools.partial(
    pl.kernel,
    mesh=_mesh,
    out_type=jax.ShapeDtypeStruct((BATCH,), jnp.float32),
    scratch_types=[
        pltpu.VMEM((SCORES_PAD,), jnp.float32),
        pltpu.VMEM((IDX_PER,), jnp.int32),
        pltpu.VMEM((ROWS_PER,), jnp.float32),
    ],
    compiler_params=pltpu.CompilerParams(needs_layout_passes=False),
)
def _pool_kernel(scores_hbm, text_hbm, out_hbm, scores_v, idx_v, out_v):
    wid = lax.axis_index("s") * 2 + lax.axis_index("c")
    base = wid * IDX_PER
    pltpu.sync_copy(text_hbm.at[pl.ds(base, IDX_PER)], idx_v)
    pltpu.sync_copy(scores_hbm, scores_v)
    row_stride = jnp.arange(LANES, dtype=jnp.int32) * SEQ
    zeros = jnp.zeros((LANES,), jnp.float32)

    for g in range(GROUPS):
        gbase = g * LANES * SEQ

        def body(l, acc, gbase=gbase):
            pos = row_stride + (gbase + l)
            idx = plsc.load_gather(idx_v, [pos])
            vals = plsc.load_gather(scores_v, [idx])
            return acc + vals

        acc = lax.fori_loop(0, SEQ, body, zeros)
        out_v[pl.ds(g * LANES, LANES)] = acc

    pltpu.sync_copy(out_v, out_hbm.at[pl.ds(wid * ROWS_PER, ROWS_PER)])


def kernel(text, emb_table, W, b):
    text_flat = text.reshape(-1).astype(jnp.int32)
    scores = _compute_scores(
        emb_table, W.reshape(1, EMBED), b.reshape(1, 1)
    ).reshape(-1)
    out = _pool_kernel(scores, text_flat)
    return out.reshape(BATCH, 1)


# speedup vs baseline: 38.7256x; 1.2663x over previous
import functools

import jax
import jax.numpy as jnp
from jax import lax
from jax.experimental import pallas as pl
from jax.experimental.pallas import tpu as pltpu
from jax.experimental.pallas import tpu_sc as plsc

VOCAB = 100000
EMBED = 128
BATCH = 4096
SEQ = 200

TC_BLOCK = 4096
N_BLOCKS = -(-VOCAB // TC_BLOCK)
SCORES_PAD = N_BLOCKS * TC_BLOCK


def _scores_body(w_ref, b_ref, emb_ref, out_ref):
    s = lax.dot_general(
        w_ref[...], emb_ref[...],
        dimension_numbers=(((0,), (1,)), ((), ())),
        preferred_element_type=jnp.float32,
    )
    out_ref[...] = ((s + b_ref[0]) * (1.0 / SEQ)).reshape(TC_BLOCK)


def _compute_scores(emb_table, W, b):
    return pl.pallas_call(
        _scores_body,
        grid=(N_BLOCKS,),
        in_specs=[
            pl.BlockSpec((EMBED, 1), lambda i: (0, 0)),
            pl.BlockSpec((1,), lambda i: (0,)),
            pl.BlockSpec((TC_BLOCK, EMBED), lambda i: (i, 0)),
        ],
        out_specs=pl.BlockSpec((TC_BLOCK,), lambda i: (i,)),
        out_shape=jax.ShapeDtypeStruct((SCORES_PAD,), jnp.float32),
    )(W, b, emb_table)


NUM_WORKERS = 32
ROWS_PER = BATCH // NUM_WORKERS
IDX_PER = ROWS_PER * SEQ
LANES = 16
GROUPS = ROWS_PER // LANES

_mesh = plsc.VectorSubcoreMesh(core_axis_name="c", subcore_axis_name="s")


@functools.partial(
    pl.kernel,
    mesh=_mesh,
    out_type=jax.ShapeDtypeStruct((BATCH,), jnp.float32),
    scratch_types=[
        pltpu.VMEM((SCORES_PAD,), jnp.float32),
        pltpu.VMEM((IDX_PER,), jnp.int32),
        pltpu.VMEM((ROWS_PER,), jnp.float32),
        pltpu.SemaphoreType.DMA,
        pltpu.SemaphoreType.DMA,
    ],
    compiler_params=pltpu.CompilerParams(needs_layout_passes=False),
)
def _pool_kernel(scores_hbm, text_hbm, out_hbm, scores_v, idx_v, out_v,
                 idx_sem, sc_sem):
    wid = lax.axis_index("s") * 2 + lax.axis_index("c")
    base = wid * IDX_PER
    idx_cp = pltpu.async_copy(text_hbm.at[pl.ds(base, IDX_PER)], idx_v, idx_sem)
    sc_cp = pltpu.async_copy(scores_hbm, scores_v, sc_sem)
    idx_cp.wait()
    sc_cp.wait()
    row_stride = jnp.arange(LANES, dtype=jnp.int32) * SEQ
    zeros = jnp.zeros((LANES,), jnp.float32)
    group_pos = [row_stride + g * LANES * SEQ for g in range(GROUPS)]

    def body(l, accs):
        out = []
        for g in range(GROUPS):
            pos = group_pos[g] + l
            idx = plsc.load_gather(idx_v, [pos])
            vals = plsc.load_gather(scores_v, [idx])
            out.append(accs[g] + vals)
        return tuple(out)

    accs = lax.fori_loop(0, SEQ, body, (zeros,) * GROUPS)
    for g in range(GROUPS):
        out_v[pl.ds(g * LANES, LANES)] = accs[g]

    pltpu.sync_copy(out_v, out_hbm.at[pl.ds(wid * ROWS_PER, ROWS_PER)])


def kernel(text, emb_table, W, b):
    text_flat = text.astype(jnp.int32).reshape(-1)
    scores = _compute_scores(emb_table, W, b)
    out = _pool_kernel(scores, text_flat)
    return out.reshape(BATCH, 1)


# speedup vs baseline: 40.3138x; 1.0410x over previous
import functools

import jax
import jax.numpy as jnp
from jax import lax
from jax.experimental import pallas as pl
from jax.experimental.pallas import tpu as pltpu
from jax.experimental.pallas import tpu_sc as plsc

VOCAB = 100000
EMBED = 128
BATCH = 4096
SEQ = 200

TC_BLOCK = 4096
N_BLOCKS = -(-VOCAB // TC_BLOCK)
SCORES_PAD = N_BLOCKS * TC_BLOCK


def _scores_body(w_ref, b_ref, emb_ref, out_ref):
    s = lax.dot_general(
        w_ref[...], emb_ref[...],
        dimension_numbers=(((0,), (1,)), ((), ())),
        preferred_element_type=jnp.float32,
    )
    out_ref[...] = ((s + b_ref[0]) * (1.0 / SEQ)).reshape(TC_BLOCK)


def _compute_scores(emb_table, W, b):
    return pl.pallas_call(
        _scores_body,
        grid=(N_BLOCKS,),
        in_specs=[
            pl.BlockSpec((EMBED, 1), lambda i: (0, 0)),
            pl.BlockSpec((1,), lambda i: (0,)),
            pl.BlockSpec((TC_BLOCK, EMBED), lambda i: (i, 0)),
        ],
        out_specs=pl.BlockSpec((TC_BLOCK,), lambda i: (i,)),
        out_shape=jax.ShapeDtypeStruct((SCORES_PAD,), jnp.float32),
    )(W, b, emb_table)


NUM_WORKERS = 32
ROWS_PER = BATCH // NUM_WORKERS
IDX_PER = ROWS_PER * SEQ
LANES = 16
GROUPS = ROWS_PER // LANES

_mesh = plsc.VectorSubcoreMesh(core_axis_name="c", subcore_axis_name="s")

_CHUNK_STARTS = tuple(16 * c for c in range(SEQ // 16)) + (SEQ - 16,)


@functools.partial(
    pl.kernel,
    mesh=_mesh,
    out_type=jax.ShapeDtypeStruct((BATCH * SEQ,), jnp.int32),
    scratch_types=[
        pltpu.VMEM((ROWS_PER, SEQ), jnp.int32),
        pltpu.VMEM((IDX_PER,), jnp.int32),
    ],
    compiler_params=pltpu.CompilerParams(needs_layout_passes=False),
)
def _flatten_kernel(text_hbm, out_hbm, t2_v, flat_v):
    wid = lax.axis_index("s") * 2 + lax.axis_index("c")
    row0 = wid * ROWS_PER
    pltpu.sync_copy(text_hbm.at[pl.ds(row0, ROWS_PER), :], t2_v)

    def body(r, carry):
        rb = r * SEQ
        for c0 in _CHUNK_STARTS:
            flat_v[pl.ds(rb + c0, LANES)] = t2_v[r, pl.ds(c0, LANES)]
        return carry

    lax.fori_loop(0, ROWS_PER, body, 0)
    pltpu.sync_copy(flat_v, out_hbm.at[pl.ds(wid * IDX_PER, IDX_PER)])


@functools.partial(
    pl.kernel,
    mesh=_mesh,
    out_type=jax.ShapeDtypeStruct((BATCH,), jnp.float32),
    scratch_types=[
        pltpu.VMEM((SCORES_PAD,), jnp.float32),
        pltpu.VMEM((IDX_PER,), jnp.int32),
        pltpu.VMEM((ROWS_PER,), jnp.float32),
        pltpu.SemaphoreType.DMA,
        pltpu.SemaphoreType.DMA,
    ],
    compiler_params=pltpu.CompilerParams(needs_layout_passes=False),
)
def _pool_kernel(scores_hbm, text_hbm, out_hbm, scores_v, idx_v, out_v,
                 idx_sem, sc_sem):
    wid = lax.axis_index("s") * 2 + lax.axis_index("c")
    base = wid * IDX_PER
    idx_cp = pltpu.async_copy(text_hbm.at[pl.ds(base, IDX_PER)], idx_v, idx_sem)
    sc_cp = pltpu.async_copy(scores_hbm, scores_v, sc_sem)
    idx_cp.wait()
    sc_cp.wait()
    row_stride = jnp.arange(LANES, dtype=jnp.int32) * SEQ
    zeros = jnp.zeros((LANES,), jnp.float32)
    group_pos = [row_stride + g * LANES * SEQ for g in range(GROUPS)]

    def body(l, accs):
        out = []
        for g in range(GROUPS):
            pos = group_pos[g] + l
            idx = plsc.load_gather(idx_v, [pos])
            vals = plsc.load_gather(scores_v, [idx])
            out.append(accs[g] + vals)
        return tuple(out)

    accs = lax.fori_loop(0, SEQ, body, (zeros,) * GROUPS)
    for g in range(GROUPS):
        out_v[pl.ds(g * LANES, LANES)] = accs[g]

    pltpu.sync_copy(out_v, out_hbm.at[pl.ds(wid * ROWS_PER, ROWS_PER)])


def kernel(text, emb_table, W, b):
    text_flat = _flatten_kernel(text.astype(jnp.int32))
    scores = _compute_scores(emb_table, W, b)
    out = _pool_kernel(scores, text_flat)
    return out.reshape(BATCH, 1)


# speedup vs baseline: 44.3844x; 1.1010x over previous
import functools

import jax
import jax.numpy as jnp
from jax import lax
from jax.experimental import pallas as pl
from jax.experimental.pallas import tpu as pltpu
from jax.experimental.pallas import tpu_sc as plsc

VOCAB = 100000
EMBED = 128
BATCH = 4096
SEQ = 200

TC_BLOCK = 8192
N_BLOCKS = -(-VOCAB // TC_BLOCK)
SCORES_PAD = 102400


def _scores_body(w_ref, b_ref, emb_ref, out_ref):
    s = lax.dot_general(
        w_ref[...], emb_ref[...],
        dimension_numbers=(((0,), (1,)), ((), ())),
        preferred_element_type=jnp.float32,
    )
    out_ref[...] = ((s + b_ref[0]) * (1.0 / SEQ)).reshape(TC_BLOCK)


def _compute_scores(emb_table, W, b):
    return pl.pallas_call(
        _scores_body,
        grid=(N_BLOCKS,),
        in_specs=[
            pl.BlockSpec((EMBED, 1), lambda i: (0, 0)),
            pl.BlockSpec((1,), lambda i: (0,)),
            pl.BlockSpec((TC_BLOCK, EMBED), lambda i: (i, 0)),
        ],
        out_specs=pl.BlockSpec((TC_BLOCK,), lambda i: (i,)),
        out_shape=jax.ShapeDtypeStruct((SCORES_PAD,), jnp.float32),
    )(W, b, emb_table)


NUM_WORKERS = 32
ROWS_PER = BATCH // NUM_WORKERS
IDX_PER = ROWS_PER * SEQ
LANES = 16
GROUPS = ROWS_PER // LANES

_mesh = plsc.VectorSubcoreMesh(core_axis_name="c", subcore_axis_name="s")

_CHUNK_STARTS = tuple(16 * c for c in range(SEQ // 16)) + (SEQ - 16,)


@functools.partial(
    pl.kernel,
    mesh=_mesh,
    out_type=jax.ShapeDtypeStruct((BATCH * SEQ,), jnp.int32),
    scratch_types=[
        pltpu.VMEM((ROWS_PER, SEQ), jnp.int32),
        pltpu.VMEM((IDX_PER,), jnp.int32),
    ],
    compiler_params=pltpu.CompilerParams(needs_layout_passes=False),
)
def _flatten_kernel(text_hbm, out_hbm, t2_v, flat_v):
    wid = lax.axis_index("s") * 2 + lax.axis_index("c")
    pltpu.sync_copy(text_hbm.at[wid], t2_v)

    def body(r, carry):
        rb = r * SEQ
        for c0 in _CHUNK_STARTS:
            flat_v[pl.ds(rb + c0, LANES)] = t2_v[r, pl.ds(c0, LANES)]
        return carry

    lax.fori_loop(0, ROWS_PER, body, 0)
    pltpu.sync_copy(flat_v, out_hbm.at[pl.ds(wid * IDX_PER, IDX_PER)])


@functools.partial(
    pl.kernel,
    mesh=_mesh,
    out_type=jax.ShapeDtypeStruct((BATCH,), jnp.float32),
    scratch_types=[
        pltpu.VMEM((SCORES_PAD,), jnp.float32),
        pltpu.VMEM((IDX_PER,), jnp.int32),
        pltpu.VMEM((ROWS_PER,), jnp.float32),
        pltpu.SemaphoreType.DMA,
        pltpu.SemaphoreType.DMA,
    ],
    compiler_params=pltpu.CompilerParams(needs_layout_passes=False),
)
def _pool_kernel(scores_hbm, text_hbm, out_hbm, scores_v, idx_v, out_v,
                 idx_sem, sc_sem):
    wid = lax.axis_index("s") * 2 + lax.axis_index("c")
    base = wid * IDX_PER
    idx_cp = pltpu.async_copy(text_hbm.at[pl.ds(base, IDX_PER)], idx_v, idx_sem)
    sc_cp = pltpu.async_copy(scores_hbm, scores_v, sc_sem)
    idx_cp.wait()
    sc_cp.wait()
    row_stride = jnp.arange(LANES, dtype=jnp.int32) * SEQ
    zeros = jnp.zeros((LANES,), jnp.float32)
    group_pos = [row_stride + g * LANES * SEQ for g in range(GROUPS)]

    def body(l, accs):
        out = []
        for g in range(GROUPS):
            pos = group_pos[g] + l
            idx = plsc.load_gather(idx_v, [pos])
            vals = plsc.load_gather(scores_v, [idx])
            out.append(accs[g] + vals)
        return tuple(out)

    accs = lax.fori_loop(0, SEQ, body, (zeros,) * GROUPS)
    for g in range(GROUPS):
        out_v[pl.ds(g * LANES, LANES)] = accs[g]

    pltpu.sync_copy(out_v, out_hbm.at[pl.ds(wid * ROWS_PER, ROWS_PER)])


def kernel(text, emb_table, W, b):
    text3d = text.astype(jnp.int32).reshape(NUM_WORKERS, ROWS_PER, SEQ)
    text_flat = _flatten_kernel(text3d)
    scores = _compute_scores(emb_table, W, b)
    out = _pool_kernel(scores, text_flat)
    return out.reshape(BATCH, 1)


# speedup vs baseline: 48.7080x; 1.0974x over previous
import functools

import jax
import jax.numpy as jnp
from jax import lax
from jax.experimental import pallas as pl
from jax.experimental.pallas import tpu as pltpu
from jax.experimental.pallas import tpu_sc as plsc

VOCAB = 100000
EMBED = 128
BATCH = 4096
SEQ = 200

HALF = 51200
TC_BLOCK = 5120
N_BLOCKS = HALF // TC_BLOCK
SCORES_PAD = 2 * HALF


def _scores_body(w_ref, b_ref, emb_lo_ref, emb_hi_ref, out_ref):
    def score_bits(emb_ref):
        s = lax.dot_general(
            w_ref[...], emb_ref[...],
            dimension_numbers=(((0,), (1,)), ((), ())),
            preferred_element_type=jnp.float32,
        )
        s = (s + b_ref[0]) * (1.0 / SEQ)
        return lax.bitcast_convert_type(s, jnp.int32) + 0x8000

    lo = jnp.right_shift(score_bits(emb_lo_ref), 16) & 0xFFFF
    hi = score_bits(emb_hi_ref) & jnp.int32(-65536)
    out_ref[...] = (hi | lo).reshape(TC_BLOCK)


def _compute_scores(emb_table, W, b):
    return pl.pallas_call(
        _scores_body,
        grid=(N_BLOCKS,),
        in_specs=[
            pl.BlockSpec((EMBED, 1), lambda i: (0, 0)),
            pl.BlockSpec((1,), lambda i: (0,)),
            pl.BlockSpec((TC_BLOCK, EMBED), lambda i: (i, 0)),
            pl.BlockSpec((TC_BLOCK, EMBED), lambda i: (i + N_BLOCKS, 0)),
        ],
        out_specs=pl.BlockSpec((TC_BLOCK,), lambda i: (i,)),
        out_shape=jax.ShapeDtypeStruct((HALF,), jnp.int32),
    )(W, b, emb_table, emb_table)


NUM_WORKERS = 32
ROWS_PER = BATCH // NUM_WORKERS
IDX_PER = ROWS_PER * SEQ
LANES = 16
GROUPS = ROWS_PER // LANES

_mesh = plsc.VectorSubcoreMesh(core_axis_name="c", subcore_axis_name="s")

_CHUNK_STARTS = tuple(16 * c for c in range(SEQ // 16)) + (SEQ - 16,)


@functools.partial(
    pl.kernel,
    mesh=_mesh,
    out_type=jax.ShapeDtypeStruct((BATCH * SEQ,), jnp.int32),
    scratch_types=[
        pltpu.VMEM((ROWS_PER, SEQ), jnp.int32),
        pltpu.VMEM((IDX_PER,), jnp.int32),
    ],
    compiler_params=pltpu.CompilerParams(needs_layout_passes=False),
)
def _flatten_kernel(text_hbm, out_hbm, t2_v, flat_v):
    wid = lax.axis_index("s") * 2 + lax.axis_index("c")
    pltpu.sync_copy(text_hbm.at[wid], t2_v)

    def body(r, carry):
        rb = r * SEQ
        for c0 in _CHUNK_STARTS:
            flat_v[pl.ds(rb + c0, LANES)] = t2_v[r, pl.ds(c0, LANES)]
        return carry

    lax.fori_loop(0, ROWS_PER, body, 0)
    pltpu.sync_copy(flat_v, out_hbm.at[pl.ds(wid * IDX_PER, IDX_PER)])


@functools.partial(
    pl.kernel,
    mesh=_mesh,
    out_type=jax.ShapeDtypeStruct((BATCH,), jnp.float32),
    scratch_types=[
        pltpu.VMEM((HALF,), jnp.int32),
        pltpu.VMEM((IDX_PER,), jnp.int32),
        pltpu.VMEM((ROWS_PER,), jnp.float32),
        pltpu.SemaphoreType.DMA,
        pltpu.SemaphoreType.DMA,
    ],
    compiler_params=pltpu.CompilerParams(needs_layout_passes=False),
)
def _pool_kernel(scores_hbm, text_hbm, out_hbm, scores_v, idx_v, out_v,
                 idx_sem, sc_sem):
    wid = lax.axis_index("s") * 2 + lax.axis_index("c")
    base = wid * IDX_PER
    idx_cp = pltpu.async_copy(text_hbm.at[pl.ds(base, IDX_PER)], idx_v, idx_sem)
    sc_cp = pltpu.async_copy(scores_hbm, scores_v, sc_sem)
    idx_cp.wait()
    sc_cp.wait()
    row_stride = jnp.arange(LANES, dtype=jnp.int32) * SEQ
    zeros = jnp.zeros((LANES,), jnp.float32)
    group_pos = [row_stride + g * LANES * SEQ for g in range(GROUPS)]

    mask_hi = jnp.int32(-65536)

    def body(l, accs):
        out = []
        for g in range(GROUPS):
            pos = group_pos[g] + l
            idx = plsc.load_gather(idx_v, [pos])
            in_hi = idx >= HALF
            k = idx - jnp.where(in_hi, HALF, 0)
            w = plsc.load_gather(scores_v, [k])
            bits = jnp.where(in_hi, w & mask_hi, w << 16)
            out.append(accs[g] + plsc.bitcast(bits, jnp.float32))
        return tuple(out)

    accs = lax.fori_loop(0, SEQ, body, (zeros,) * GROUPS)
    for g in range(GROUPS):
        out_v[pl.ds(g * LANES, LANES)] = accs[g]

    pltpu.sync_copy(out_v, out_hbm.at[pl.ds(wid * ROWS_PER, ROWS_PER)])


def kernel(text, emb_table, W, b):
    text3d = text.astype(jnp.int32).reshape(NUM_WORKERS, ROWS_PER, SEQ)
    text_flat = _flatten_kernel(text3d)
    scores = _compute_scores(emb_table, W, b)
    out = _pool_kernel(scores, text_flat)
    return out.reshape(BATCH, 1)


# speedup vs baseline: 48.7289x; 1.0004x over previous
import functools

import jax
import jax.numpy as jnp
from jax import lax
from jax.experimental import pallas as pl
from jax.experimental.pallas import tpu as pltpu
from jax.experimental.pallas import tpu_sc as plsc

VOCAB = 100000
EMBED = 128
BATCH = 4096
SEQ = 200

HALF = 51200
TC_BLOCK = 5120
N_BLOCKS = HALF // TC_BLOCK
SCORES_PAD = 2 * HALF


def _scores_body(w_ref, b_ref, emb_lo_ref, emb_hi_ref, out_ref):
    def score_bits(emb_ref):
        s = lax.dot_general(
            w_ref[...], emb_ref[...],
            dimension_numbers=(((0,), (1,)), ((), ())),
            preferred_element_type=jnp.float32,
        )
        s = (s + b_ref[0]) * (1.0 / SEQ)
        return lax.bitcast_convert_type(s, jnp.int32) + 0x8000

    lo = jnp.right_shift(score_bits(emb_lo_ref), 16) & 0xFFFF
    hi = score_bits(emb_hi_ref) & jnp.int32(-65536)
    out_ref[...] = (hi | lo).reshape(TC_BLOCK)


def _compute_scores(emb_table, W, b):
    return pl.pallas_call(
        _scores_body,
        grid=(N_BLOCKS,),
        in_specs=[
            pl.BlockSpec((EMBED, 1), lambda i: (0, 0)),
            pl.BlockSpec((1,), lambda i: (0,)),
            pl.BlockSpec((TC_BLOCK, EMBED), lambda i: (i, 0)),
            pl.BlockSpec((TC_BLOCK, EMBED), lambda i: (i + N_BLOCKS, 0)),
        ],
        out_specs=pl.BlockSpec((TC_BLOCK,), lambda i: (i,)),
        out_shape=jax.ShapeDtypeStruct((HALF,), jnp.int32),
    )(W, b, emb_table, emb_table)


NUM_WORKERS = 32
ROWS_PER = BATCH // NUM_WORKERS
IDX_PER = ROWS_PER * SEQ
LANES = 16
GROUPS = ROWS_PER // LANES

_mesh = plsc.VectorSubcoreMesh(core_axis_name="c", subcore_axis_name="s")

_CHUNK_STARTS = tuple(16 * c for c in range(SEQ // 16)) + (SEQ - 16,)


@functools.partial(
    pl.kernel,
    mesh=_mesh,
    out_type=jax.ShapeDtypeStruct((BATCH * SEQ,), jnp.int32),
    scratch_types=[
        pltpu.VMEM((ROWS_PER, SEQ), jnp.int32),
        pltpu.VMEM((IDX_PER,), jnp.int32),
    ],
    compiler_params=pltpu.CompilerParams(needs_layout_passes=False),
)
def _flatten_kernel(text_hbm, out_hbm, t2_v, flat_v):
    wid = lax.axis_index("s") * 2 + lax.axis_index("c")
    pltpu.sync_copy(text_hbm.at[pl.ds(wid * ROWS_PER, ROWS_PER), :], t2_v)

    def body(r, carry):
        rb = r * SEQ
        for c0 in _CHUNK_STARTS:
            flat_v[pl.ds(rb + c0, LANES)] = t2_v[r, pl.ds(c0, LANES)]
        return carry

    lax.fori_loop(0, ROWS_PER, body, 0)
    pltpu.sync_copy(flat_v, out_hbm.at[pl.ds(wid * IDX_PER, IDX_PER)])


@functools.partial(
    pl.kernel,
    mesh=_mesh,
    out_type=jax.ShapeDtypeStruct((BATCH,), jnp.float32),
    scratch_types=[
        pltpu.VMEM((HALF,), jnp.int32),
        pltpu.VMEM((IDX_PER,), jnp.int32),
        pltpu.VMEM((ROWS_PER,), jnp.float32),
        pltpu.SemaphoreType.DMA,
        pltpu.SemaphoreType.DMA,
    ],
    compiler_params=pltpu.CompilerParams(needs_layout_passes=False),
)
def _pool_kernel(scores_hbm, text_hbm, out_hbm, scores_v, idx_v, out_v,
                 idx_sem, sc_sem):
    wid = lax.axis_index("s") * 2 + lax.axis_index("c")
    base = wid * IDX_PER
    idx_cp = pltpu.async_copy(text_hbm.at[pl.ds(base, IDX_PER)], idx_v, idx_sem)
    sc_cp = pltpu.async_copy(scores_hbm, scores_v, sc_sem)
    idx_cp.wait()
    sc_cp.wait()
    row_stride = jnp.arange(LANES, dtype=jnp.int32) * SEQ
    zeros = jnp.zeros((LANES,), jnp.float32)
    group_pos = [row_stride + g * LANES * SEQ for g in range(GROUPS)]

    mask_hi = jnp.int32(-65536)

    def body(l, accs):
        out = []
        for g in range(GROUPS):
            pos = group_pos[g] + l
            idx = plsc.load_gather(idx_v, [pos])
            in_hi = idx >= HALF
            k = idx - jnp.where(in_hi, HALF, 0)
            w = plsc.load_gather(scores_v, [k])
            bits = jnp.where(in_hi, w & mask_hi, w << 16)
            out.append(accs[g] + plsc.bitcast(bits, jnp.float32))
        return tuple(out)

    accs = lax.fori_loop(0, SEQ, body, (zeros,) * GROUPS)
    for g in range(GROUPS):
        out_v[pl.ds(g * LANES, LANES)] = accs[g]

    pltpu.sync_copy(out_v, out_hbm.at[pl.ds(wid * ROWS_PER, ROWS_PER)])


def kernel(text, emb_table, W, b):
    text_flat = _flatten_kernel(text.astype(jnp.int32))
    scores = _compute_scores(emb_table, W, b)
    out = _pool_kernel(scores, text_flat)
    return out.reshape(BATCH, 1)
